# Initial kernel scaffold; baseline (speedup 1.0000x reference)
import jax
import jax.numpy as jnp
from jax.experimental import pallas as pl


def kernel(x, router_w, router_b, W1, b1, W2, b2, in_proj_w, in_proj_b, out_proj_w, out_proj_b, ln1_w, ln1_b, ln2_w, ln2_b):
    raise NotImplementedError("write your pallas kernel here")


# speedup vs baseline: 3.2219x; 3.2219x over previous
import functools

import jax
import jax.numpy as jnp
from jax import lax
from jax.experimental import pallas as pl
from jax.experimental.pallas import tpu as pltpu

B, S, D, E, K, H = 1, 2048, 768, 8, 2, 12
F = 4 * D
DH = D // H

SBLK = 512
NS = S // SBLK


def _pool_dense_kernel(x_ref, rw_ref, rb_ref, w1_ref, b1_ref, w2_ref, b2_ref,
                       out_ref):
    e = pl.program_id(0)
    s = pl.program_id(1)
    xb = x_ref[...]
    logits = jnp.dot(xb, rw_ref[...], preferred_element_type=jnp.float32)
    logits = logits + rb_ref[...]
    ecol = lax.broadcasted_iota(jnp.int32, (SBLK, E), 1)
    cnt = jnp.zeros((SBLK, E), jnp.float32)
    for j in range(E):
        lj = logits[:, j:j + 1]
        beats = (lj > logits) | ((lj == logits) & (j < ecol))
        cnt = cnt + beats.astype(jnp.float32)
    mask_all = cnt < float(K)
    sel = jnp.where(mask_all & (ecol == e), 1.0, 0.0)
    mask_e = jnp.sum(sel, axis=1, keepdims=True)

    h = jnp.dot(xb, w1_ref[0], preferred_element_type=jnp.float32)
    h = h + b1_ref[0]
    h = 0.5 * h * (1.0 + lax.erf(h * (2.0 ** -0.5)))
    y = jnp.dot(h, w2_ref[0], preferred_element_type=jnp.float32)
    y = y + b2_ref[0]
    contrib = (mask_e * (1.0 / K)) * y

    sl = pl.ds(s * SBLK, SBLK)

    @pl.when(e == 0)
    def _():
        out_ref[sl, :] = contrib

    @pl.when(e > 0)
    def _():
        out_ref[sl, :] = out_ref[sl, :] + contrib


def _qkv_kernel(no_ref, w_ref, b_ref, out_ref):
    out_ref[...] = lax.dot_general(
        no_ref[...], w_ref[...], (((1,), (1,)), ((), ())),
        preferred_element_type=jnp.float32) + b_ref[...]


def _attn_kernel(q_ref, k_ref, v_ref, out_ref):
    for i in range(2):
        cols = slice(i * DH, (i + 1) * DH)
        q = q_ref[:, cols]
        k = k_ref[:, cols]
        v = v_ref[:, cols]
        s = lax.dot_general(q, k, (((1,), (1,)), ((), ())),
                            preferred_element_type=jnp.float32)
        s = s * (1.0 / (DH ** 0.5))
        m = jnp.max(s, axis=1, keepdims=True)
        p = jnp.exp(s - m)
        p = p / jnp.sum(p, axis=1, keepdims=True)
        out_ref[:, cols] = jnp.dot(p, v, preferred_element_type=jnp.float32)


def _layernorm(x, w, b, eps=1e-5):
    mu = jnp.mean(x, axis=-1, keepdims=True)
    xc = x - mu
    var = jnp.mean(xc * xc, axis=-1, keepdims=True)
    return xc * jax.lax.rsqrt(var + eps) * w + b


def _final_kernel(x_ref, no_ref, ao_ref, w_ref, b_ref, l1w_ref, l1b_ref,
                  l2w_ref, l2b_ref, out_ref):
    attn_out = lax.dot_general(
        ao_ref[...], w_ref[...], (((1,), (1,)), ((), ())),
        preferred_element_type=jnp.float32) + b_ref[...]
    x1 = _layernorm(x_ref[...] + attn_out, l1w_ref[...], l1b_ref[...])
    out_ref[...] = _layernorm(x1 + no_ref[...], l2w_ref[...], l2b_ref[...])


def kernel(x, router_w, router_b, W1, b1, W2, b2, in_proj_w, in_proj_b,
           out_proj_w, out_proj_b, ln1_w, ln1_b, ln2_w, ln2_b):
    x2 = x.reshape(S, D)
    rb2 = router_b.reshape(1, E)

    neuron_outputs = pl.pallas_call(
        _pool_dense_kernel,
        grid=(E, NS),
        in_specs=[
            pl.BlockSpec((SBLK, D), lambda e, s: (s, 0)),
            pl.BlockSpec((D, E), lambda e, s: (0, 0)),
            pl.BlockSpec((1, E), lambda e, s: (0, 0)),
            pl.BlockSpec((1, D, F), lambda e, s: (e, 0, 0)),
            pl.BlockSpec((1, 1, F), lambda e, s: (e, 0, 0)),
            pl.BlockSpec((1, F, D), lambda e, s: (e, 0, 0)),
            pl.BlockSpec((1, 1, D), lambda e, s: (e, 0, 0)),
        ],
        out_specs=pl.BlockSpec((S, D), lambda e, s: (0, 0)),
        out_shape=jax.ShapeDtypeStruct((S, D), jnp.float32),
    )(x2, router_w, rb2, W1, b1.reshape(E, 1, F), W2, b2.reshape(E, 1, D))

    qkv = pl.pallas_call(
        _qkv_kernel,
        out_shape=jax.ShapeDtypeStruct((S, 3 * D), jnp.float32),
    )(neuron_outputs, in_proj_w, in_proj_b.reshape(1, 3 * D))

    ao = pl.pallas_call(
        _attn_kernel,
        grid=(H // 2,),
        in_specs=[
            pl.BlockSpec((S, 2 * DH), lambda h: (0, h)),
            pl.BlockSpec((S, 2 * DH), lambda h: (0, H // 2 + h)),
            pl.BlockSpec((S, 2 * DH), lambda h: (0, H + h)),
        ],
        out_specs=pl.BlockSpec((S, 2 * DH), lambda h: (0, h)),
        out_shape=jax.ShapeDtypeStruct((S, D), jnp.float32),
    )(qkv, qkv, qkv)

    out = pl.pallas_call(
        _final_kernel,
        out_shape=jax.ShapeDtypeStruct((S, D), jnp.float32),
    )(x2, neuron_outputs, ao, out_proj_w, out_proj_b.reshape(1, D),
      ln1_w.reshape(1, D), ln1_b.reshape(1, D),
      ln2_w.reshape(1, D), ln2_b.reshape(1, D))

    return out.reshape(B, S, D)
